# baseline (speedup 1.0000x reference)
import functools

import jax
import jax.numpy as jnp
from jax import lax
from jax.experimental import pallas as pl
from jax.experimental.pallas import tpu as pltpu
from jax.experimental.pallas import tpu_sc as plsc

N = 10000
K = 16
E = 160000
NPRO = 2048
EPRO = 32768
M = 65536
H = 128
VOCAB = 64
PDIM = 1280

NPAD = 10240
CH = 128
NWORK = 32
EPAD = 163840
CPW = EPAD // (NWORK * CH)
ROWS_PER_TILE = NPAD // 16
OUT_ROWS_PER_TILE = N // 16


def _dot(a, b):
    return jnp.dot(a, b, preferred_element_type=jnp.float32)


def _node_kernel(ac_ref, ad_ref, x_ref,
                 emb_ref, w1a_ref, b1a_ref, w2a_ref, b2a_ref,
                 w1b_ref, b1b_ref, w2b_ref, b2b_ref,
                 w1c_ref, b1c_ref, w2c_ref, b2c_ref,
                 wf_ref, bf_ref, feat_ref):
    B = ac_ref.shape[0]
    T = _dot(emb_ref[...], w1a_ref[0:32, :])
    w1last = w1a_ref[32:33, :]
    acc = jnp.zeros((B, H), jnp.float32)
    iot = lax.broadcasted_iota(jnp.int32, (B, VOCAB), 1)
    for k in range(K):
        ack = ac_ref[:, k:k + 1]
        oh = (ack == iot).astype(jnp.float32)
        l1 = _dot(oh, T) + (1.0 / ad_ref[:, k:k + 1]) * w1last + b1a_ref[...]
        acc = acc + jnp.maximum(l1, 0.0)
    h = _dot(acc, w2a_ref[...]) + K * b2a_ref[...]
    h = _dot(jnp.maximum(_dot(h, w1b_ref[...]) + b1b_ref[...], 0.0),
             w2b_ref[...]) + b2b_ref[...]
    geom = x_ref[:, 0:1]
    l = jnp.maximum(_dot(h, w1c_ref[0:H, :]) + geom * w1c_ref[H:H + 1, :]
                    + b1c_ref[...], 0.0)
    l = _dot(l, w2c_ref[...]) + b2c_ref[...]
    feat_ref[...] = _dot(l, wf_ref[...]) + bf_ref[...]


def _node_encoder(atom_cat, atom_dist, x, p):
    B = 400
    grid = (N // B,)
    full = lambda arr: pl.BlockSpec(arr.shape, lambda i: (0,) * arr.ndim)
    row = lambda c: pl.BlockSpec((B, c), lambda i: (i, 0))
    emb = p['emb']
    w1a, b1a, w2a, b2a = p['atom_a']
    w1b, b1b, w2b, b2b = p['atom_b']
    w1c, b1c, w2c, b2c = p['chem']
    wf, bf = p['feat_scale']
    b1a, b2a, b1b, b2b, b1c, b2c, bf = (
        v.reshape(1, H) for v in (b1a, b2a, b1b, b2b, b1c, b2c, bf))
    args = (atom_cat, atom_dist, x, emb, w1a, b1a, w2a, b2a,
            w1b, b1b, w2b, b2b, w1c, b1c, w2c, b2c, wf, bf)
    in_specs = [row(K), row(K), row(x.shape[1])] + [full(a) for a in args[3:]]
    return pl.pallas_call(
        _node_kernel,
        out_shape=jax.ShapeDtypeStruct((N, H), jnp.float32),
        grid=grid,
        in_specs=in_specs,
        out_specs=pl.BlockSpec((B, H), lambda i: (i, 0)),
    )(*args)


NPHASE = 5
PC = CPW // NPHASE
SLOW_CID = 1
SLOWW = 24
FASTW = 56


def _gine_sc_body(feat_hbm, pos16_hbm, src_hbm, dst_hbm, attr_hbm, wpack_hbm,
                  out_hbm,
                  srcb, dstb, attrb, distb, attr1d,
                  featb0, featb1, spos0, spos1, dpos0, dpos1, wpk_v, agg_sh,
                  gsem0, gsem1, ssem0, ssem1):
    cid = lax.axis_index("c")
    sid = lax.axis_index("s")
    wid = sid * 2 + cid
    featbs = (featb0, featb1)
    sposs = (spos0, spos1)
    dposs = (dpos0, dpos1)
    gsems = (gsem0, gsem1)
    pltpu.sync_copy(wpack_hbm, wpk_v)

    def zrow(i, c):
        for j in range(8):
            featb0[i, pl.ds(j * 16, 16)] = jnp.zeros((16,), jnp.float32)
        return c
    lax.fori_loop(0, CH, zrow, 0)
    for z in range(ROWS_PER_TILE // CH):
        pltpu.sync_copy(featb0, agg_sh.at[pl.ds(sid * ROWS_PER_TILE + z * CH, CH)])
    plsc.subcore_barrier()

    wvecs = [(wpk_v[0, pl.ds(j * 16, 16)],
              wpk_v[1, pl.ds(j * 16, 16)],
              wpk_v[2, pl.ds(j * 16, 16)]) for j in range(8)]
    lane = lax.iota(jnp.int32, 16)
    zc = lane * 0
    oc = zc + 1
    tc = zc + 2

    def gather_issue(t, b):
        pltpu.async_copy(feat_hbm.at[srcb.at[t]], featbs[b], gsems[b])
        pltpu.async_copy(pos16_hbm.at[srcb.at[t]], sposs[b], gsems[b])
        pltpu.async_copy(pos16_hbm.at[dstb.at[t]], dposs[b], gsems[b])

    def gather_wait(t, b):
        pltpu.make_async_copy(feat_hbm.at[srcb.at[t]], featbs[b],
                              gsems[b]).wait()
        pltpu.make_async_copy(pos16_hbm.at[srcb.at[t]], sposs[b],
                              gsems[b]).wait()
        pltpu.make_async_copy(pos16_hbm.at[dstb.at[t]], dposs[b],
                              gsems[b]).wait()

    def compute_chunk(t, b):
        fb = featbs[b]
        sp = sposs[b]
        dp = dposs[b]
        for i in range(8):
            sl16 = pl.ds(i * 16, 16)
            rows = lane + i * 16
            dx = (plsc.load_gather(sp, [rows, zc])
                  - plsc.load_gather(dp, [rows, zc]))
            dy = (plsc.load_gather(sp, [rows, oc])
                  - plsc.load_gather(dp, [rows, oc]))
            dz = (plsc.load_gather(sp, [rows, tc])
                  - plsc.load_gather(dp, [rows, tc]))
            r2 = dx * dx + dy * dy + dz * dz
            bi = plsc.bitcast(r2, jnp.int32)
            y = plsc.bitcast(jnp.int32(0x1FBD1DF5)
                             + lax.shift_right_logical(bi, 1), jnp.float32)
            y = 0.5 * (y + r2 / y)
            y = 0.5 * (y + r2 / y)
            y = 0.5 * (y + r2 / y)
            distb[sl16] = jnp.where(r2 > 0.0, y, 0.0)
            attr1d[sl16] = attrb[t, sl16]
        def edge_body(e, c2):
            d = distb[pl.ds(e, 16)][0]
            a = attr1d[pl.ds(e, 16)][0]
            for j in range(8):
                sl = pl.ds(j * 16, 16)
                wd, wa, wb = wvecs[j]
                fb[e, sl] = jnp.maximum(fb[e, sl] + d * wd + a * wa + wb, 0.0)
            return c2
        lax.fori_loop(0, CH, edge_body, 0)

    nphases = jnp.where(cid == SLOW_CID, SLOWW // PC, FASTW // PC)
    start = jnp.where(cid == SLOW_CID, sid * SLOWW, 16 * SLOWW + sid * FASTW)

    def phase_body(ph, c0):
        base = start + ph * PC
        pltpu.sync_copy(src_hbm.at[pl.ds(base, PC)], srcb)
        pltpu.sync_copy(dst_hbm.at[pl.ds(base, PC)], dstb)
        pltpu.sync_copy(attr_hbm.at[pl.ds(base, PC)], attrb)
        gather_issue(0, 0)

        def outer_body(tt, c):
            for b in range(2):
                t = tt * 2 + b
                ob = 1 - b

                @pl.when(t <= PC - 2)
                def _():
                    gather_issue(t + 1, ob)

                gather_wait(t, b)
                compute_chunk(t, b)
                pltpu.sync_copy(featbs[b], agg_sh.at[dstb.at[t]], add=True)
            return c
        lax.fori_loop(0, PC // 2, outer_body, 0)
        return c0
    lax.fori_loop(0, nphases, phase_body, 0)
    plsc.subcore_barrier()
    pltpu.sync_copy(agg_sh.at[pl.ds(sid * ROWS_PER_TILE, ROWS_PER_TILE)],
                    out_hbm.at[cid, pl.ds(sid * ROWS_PER_TILE, ROWS_PER_TILE)])


def _gine_aggregate(feat, pos16, src_pad, dst_pad, attr_pad, wpack):
    mesh = plsc.VectorSubcoreMesh(core_axis_name="c", subcore_axis_name="s")
    fn = functools.partial(
        pl.kernel,
        out_type=jax.ShapeDtypeStruct((2, NPAD, H), jnp.float32),
        mesh=mesh,
        scratch_types=[
            pltpu.VMEM((PC, CH), jnp.int32),
            pltpu.VMEM((PC, CH), jnp.int32),
            pltpu.VMEM((PC, CH), jnp.float32),
            pltpu.VMEM((CH + 16,), jnp.float32),
            pltpu.VMEM((CH + 16,), jnp.float32),
            pltpu.VMEM((CH, H), jnp.float32),
            pltpu.VMEM((CH, H), jnp.float32),
            pltpu.VMEM((CH, 16), jnp.float32),
            pltpu.VMEM((CH, 16), jnp.float32),
            pltpu.VMEM((CH, 16), jnp.float32),
            pltpu.VMEM((CH, 16), jnp.float32),
            pltpu.VMEM((3, H), jnp.float32),
            pltpu.VMEM_SHARED((NPAD, H), jnp.float32),
            pltpu.SemaphoreType.DMA,
            pltpu.SemaphoreType.DMA,
            pltpu.SemaphoreType.DMA,
            pltpu.SemaphoreType.DMA,
        ],
        compiler_params=pltpu.CompilerParams(needs_layout_passes=False, use_tc_tiling_on_sc=False),
    )(_gine_sc_body)
    return fn(feat, pos16, src_pad.reshape(EPAD // CH, CH),
              dst_pad.reshape(EPAD // CH, CH),
              attr_pad.reshape(EPAD // CH, CH), wpack)


def _gine_mlp_kernel(feat_ref, agg_ref, w1_ref, b1_ref, w2_ref, b2_ref,
                     wp_ref, bp_ref, feat2_ref, pred_ref):
    f = feat_ref[...]
    z = f + agg_ref[0] + agg_ref[1]
    o = _dot(jnp.maximum(_dot(z, w1_ref[...]) + b1_ref[...], 0.0),
             w2_ref[...]) + b2_ref[...] + f
    feat2_ref[...] = o
    p = _dot(o, wp_ref[...]) + bp_ref[...]
    pred_ref[...] = 1.0 / (1.0 + jnp.exp(-p))


def _gine_mlp(feat, agg2, p):
    B = 400
    grid = (N // B,)
    w1, b1, w2, b2 = p['gine']
    wp, bp = p['pred']
    b1 = b1.reshape(1, H)
    b2 = b2.reshape(1, H)
    bp = bp.reshape(1, 1)
    full = lambda arr: pl.BlockSpec(arr.shape, lambda i: (0,) * arr.ndim)
    return pl.pallas_call(
        _gine_mlp_kernel,
        out_shape=(jax.ShapeDtypeStruct((N, H), jnp.float32),
                   jax.ShapeDtypeStruct((N, 1), jnp.float32)),
        grid=grid,
        in_specs=[pl.BlockSpec((B, H), lambda i: (i, 0)),
                  pl.BlockSpec((2, B, H), lambda i: (0, i, 0)),
                  full(w1), full(b1), full(w2), full(b2), full(wp), full(bp)],
        out_specs=(pl.BlockSpec((B, H), lambda i: (i, 0)),
                   pl.BlockSpec((B, 1), lambda i: (i, 0))),
    )(feat, agg2, w1, b1, w2, b2, wp, bp)


def _pro_kernel(pe_ref, w_ref, b_ref, g_ref, bl_ref, out_ref):
    ph = _dot(pe_ref[...], w_ref[...]) + b_ref[...]
    mu = jnp.mean(ph, axis=-1, keepdims=True)
    d = ph - mu
    var = jnp.mean(d * d, axis=-1, keepdims=True)
    out_ref[...] = d * lax.rsqrt(var + 1e-5) * g_ref[...] + bl_ref[...]


def _pro_out(pro_emb, p):
    B = 256
    w, b = p['prot_lin']
    g, bl = p['ln']
    b = b.reshape(1, H)
    g = g.reshape(1, H)
    bl = bl.reshape(1, H)
    full = lambda arr: pl.BlockSpec(arr.shape, lambda i: (0,) * arr.ndim)
    return pl.pallas_call(
        _pro_kernel,
        out_shape=jax.ShapeDtypeStruct((NPRO, H), jnp.float32),
        grid=(NPRO // B,),
        in_specs=[pl.BlockSpec((B, PDIM), lambda i: (i, 0)),
                  full(w), full(b), full(g), full(bl)],
        out_specs=pl.BlockSpec((B, H), lambda i: (i, 0)),
    )(pro_emb, w, b, g, bl)


def _cross_kernel(r_ref, w_ref, b_ref, out_ref):
    out_ref[...] = (1.0 / r_ref[...]) * w_ref[...] + b_ref[...]


def _cross_all(r_all, p):
    B = 2048
    w, b = p['cross_lin']
    b = b.reshape(1, H)
    R = r_all.shape[0]
    full = lambda arr: pl.BlockSpec(arr.shape, lambda i: (0,) * arr.ndim)
    return pl.pallas_call(
        _cross_kernel,
        out_shape=jax.ShapeDtypeStruct((R, H), jnp.float32),
        grid=(R // B,),
        in_specs=[pl.BlockSpec((B, 1), lambda i: (i, 0)), full(w), full(b)],
        out_specs=pl.BlockSpec((B, H), lambda i: (i, 0)),
    )(r_all, w, b)


def _merge_kernel(pe_ref, mp_ref, out_ref):
    out_ref[:, 0:EPRO] = pe_ref[...]
    out_ref[0:1, EPRO:EPRO + M] = mp_ref[0:1, :]
    out_ref[1:2, EPRO:EPRO + M] = mp_ref[1:2, :] + NPRO
    out_ref[0:1, EPRO + M:EPRO + 2 * M] = mp_ref[1:2, :] + NPRO
    out_ref[1:2, EPRO + M:EPRO + 2 * M] = mp_ref[0:1, :]


def _merge(pro_edge, mp):
    return pl.pallas_call(
        _merge_kernel,
        out_shape=jax.ShapeDtypeStruct((2, EPRO + 2 * M), jnp.int32),
    )(pro_edge, mp)


def kernel(params, pro_emb, pos, atom_cat, atom_dist, x, edge_index, edge_attr,
           merge_pro_vertex_edge, pro_edge, prot_dist, prot_ind):
    p = params
    feat = _node_encoder(atom_cat, atom_dist, x, p)

    pos16 = jnp.zeros((NPAD, 16), jnp.float32).at[:N, 0:3].set(pos)
    npadE = EPAD - E
    src_pad = jnp.concatenate([edge_index[0], jnp.zeros((npadE,), jnp.int32)])
    dst_pad = jnp.concatenate(
        [edge_index[1], N + (jnp.arange(npadE, dtype=jnp.int32) % (NPAD - N))])
    attr_pad = jnp.concatenate([edge_attr, jnp.zeros((npadE,), jnp.float32)])
    we, be = p['edge_lin']
    wpack = jnp.stack([we[0], we[1], be], 0)
    agg2 = _gine_aggregate(feat, pos16, src_pad, dst_pad, attr_pad, wpack)

    feat2, pred2d = _gine_mlp(feat, agg2, p)
    pred = pred2d[:, 0]

    pro_out = _pro_out(pro_emb, p)

    r_all = jnp.concatenate([prot_ind, prot_dist, prot_dist], axis=0)
    cross_all = _cross_all(r_all, p)

    merge = _merge(pro_edge, merge_pro_vertex_edge)
    return (pro_out, pred, feat2, merge, cross_all)


# speedup vs baseline: 1.1726x; 1.1726x over previous
import functools

import jax
import jax.numpy as jnp
from jax import lax
from jax.experimental import pallas as pl
from jax.experimental.pallas import tpu as pltpu
from jax.experimental.pallas import tpu_sc as plsc

N = 10000
K = 16
E = 160000
NPRO = 2048
EPRO = 32768
M = 65536
H = 128
VOCAB = 64
PDIM = 1280

NPAD = 10240
CH = 128
NWORK = 32
EPAD = 163840
CPW = EPAD // (NWORK * CH)
ROWS_PER_TILE = NPAD // 16
OUT_ROWS_PER_TILE = N // 16


def _dot(a, b):
    return jnp.dot(a, b, preferred_element_type=jnp.float32)


def _node_kernel(ac_ref, ad_ref, x_ref,
                 emb_ref, w1a_ref, b1a_ref, w2a_ref, b2a_ref,
                 w1b_ref, b1b_ref, w2b_ref, b2b_ref,
                 w1c_ref, b1c_ref, w2c_ref, b2c_ref,
                 wf_ref, bf_ref, feat_ref):
    B = ac_ref.shape[0]
    T = _dot(emb_ref[...], w1a_ref[0:32, :])
    w1last = w1a_ref[32:33, :]
    acc = jnp.zeros((B, H), jnp.float32)
    iot = lax.broadcasted_iota(jnp.int32, (B, VOCAB), 1)
    for k in range(K):
        ack = ac_ref[:, k:k + 1]
        oh = (ack == iot).astype(jnp.float32)
        l1 = _dot(oh, T) + (1.0 / ad_ref[:, k:k + 1]) * w1last + b1a_ref[...]
        acc = acc + jnp.maximum(l1, 0.0)
    h = _dot(acc, w2a_ref[...]) + K * b2a_ref[...]
    h = _dot(jnp.maximum(_dot(h, w1b_ref[...]) + b1b_ref[...], 0.0),
             w2b_ref[...]) + b2b_ref[...]
    geom = x_ref[:, 0:1]
    l = jnp.maximum(_dot(h, w1c_ref[0:H, :]) + geom * w1c_ref[H:H + 1, :]
                    + b1c_ref[...], 0.0)
    l = _dot(l, w2c_ref[...]) + b2c_ref[...]
    feat_ref[...] = _dot(l, wf_ref[...]) + bf_ref[...]


def _node_encoder(atom_cat, atom_dist, x, p):
    B = 400
    grid = (N // B,)
    full = lambda arr: pl.BlockSpec(arr.shape, lambda i: (0,) * arr.ndim)
    row = lambda c: pl.BlockSpec((B, c), lambda i: (i, 0))
    emb = p['emb']
    w1a, b1a, w2a, b2a = p['atom_a']
    w1b, b1b, w2b, b2b = p['atom_b']
    w1c, b1c, w2c, b2c = p['chem']
    wf, bf = p['feat_scale']
    b1a, b2a, b1b, b2b, b1c, b2c, bf = (
        v.reshape(1, H) for v in (b1a, b2a, b1b, b2b, b1c, b2c, bf))
    args = (atom_cat, atom_dist, x, emb, w1a, b1a, w2a, b2a,
            w1b, b1b, w2b, b2b, w1c, b1c, w2c, b2c, wf, bf)
    in_specs = [row(K), row(K), row(x.shape[1])] + [full(a) for a in args[3:]]
    return pl.pallas_call(
        _node_kernel,
        out_shape=jax.ShapeDtypeStruct((N, H), jnp.float32),
        grid=grid,
        in_specs=in_specs,
        out_specs=pl.BlockSpec((B, H), lambda i: (i, 0)),
    )(*args)


NPHASE = 5
PC = CPW // NPHASE
SLOW_CID = 1
SLOWW = 40
FASTW = 40


def _gine_sc_body(feat_hbm, pos16_hbm, src_hbm, dst_hbm, attr_hbm, wpack_hbm,
                  out_hbm,
                  srcb, dstb, attrb, distb, attr1d,
                  featb0, featb1, msgb, spos0, spos1, dpos0, dpos1,
                  wpk_v, agg_sh,
                  gsem0, gsem1, ssem0, ssem1):
    cid = lax.axis_index("c")
    sid = lax.axis_index("s")
    wid = sid * 2 + cid
    featbs = (featb0, featb1)
    sposs = (spos0, spos1)
    dposs = (dpos0, dpos1)
    gsems = (gsem0, gsem1)
    pltpu.sync_copy(wpack_hbm, wpk_v)

    def zrow(i, c):
        for j in range(8):
            msgb[i, pl.ds(j * 16, 16)] = jnp.zeros((16,), jnp.float32)
        return c
    lax.fori_loop(0, CH, zrow, 0)
    for z in range(ROWS_PER_TILE // CH):
        pltpu.sync_copy(msgb, agg_sh.at[pl.ds(sid * ROWS_PER_TILE + z * CH, CH)])
    plsc.subcore_barrier()

    wvecs = [(wpk_v[0, pl.ds(j * 16, 16)],
              wpk_v[1, pl.ds(j * 16, 16)],
              wpk_v[2, pl.ds(j * 16, 16)]) for j in range(8)]
    lane = lax.iota(jnp.int32, 16)
    zc = lane * 0
    oc = zc + 1
    tc = zc + 2

    def gather_issue(t, b):
        pltpu.async_copy(feat_hbm.at[srcb.at[t]], featbs[b], gsems[b])
        pltpu.async_copy(pos16_hbm.at[srcb.at[t]], sposs[b], gsems[b])
        pltpu.async_copy(pos16_hbm.at[dstb.at[t]], dposs[b], gsems[b])

    def gather_wait(t, b):
        pltpu.make_async_copy(feat_hbm.at[srcb.at[t]], featbs[b],
                              gsems[b]).wait()
        pltpu.make_async_copy(pos16_hbm.at[srcb.at[t]], sposs[b],
                              gsems[b]).wait()
        pltpu.make_async_copy(pos16_hbm.at[dstb.at[t]], dposs[b],
                              gsems[b]).wait()

    def compute_chunk(t, b, msgb):
        fb = featbs[b]
        sp = sposs[b]
        dp = dposs[b]
        for i in range(8):
            sl16 = pl.ds(i * 16, 16)
            rows = lane + i * 16
            dx = (plsc.load_gather(sp, [rows, zc])
                  - plsc.load_gather(dp, [rows, zc]))
            dy = (plsc.load_gather(sp, [rows, oc])
                  - plsc.load_gather(dp, [rows, oc]))
            dz = (plsc.load_gather(sp, [rows, tc])
                  - plsc.load_gather(dp, [rows, tc]))
            r2 = dx * dx + dy * dy + dz * dz
            bi = plsc.bitcast(r2, jnp.int32)
            y = plsc.bitcast(jnp.int32(0x1FBD1DF5)
                             + lax.shift_right_logical(bi, 1), jnp.float32)
            y = 0.5 * (y + r2 / y)
            y = 0.5 * (y + r2 / y)
            y = 0.5 * (y + r2 / y)
            distb[sl16] = jnp.where(r2 > 0.0, y, 0.0)
            attr1d[sl16] = attrb[t, sl16]
        def edge_body(e, c2):
            d = distb[pl.ds(e, 16)][0]
            a = attr1d[pl.ds(e, 16)][0]
            for q in range(4):
                w = fb[e, pl.ds(q * 16, 16)]
                lo, hi = plsc.unpack(plsc.bitcast(w, jnp.bfloat16),
                                     format=plsc.PackFormat.INTERLEAVED)
                for h, v in ((2 * q, lo), (2 * q + 1, hi)):
                    sl = pl.ds(h * 16, 16)
                    wd, wa, wb = wvecs[h]
                    msgb[e, sl] = jnp.maximum(
                        v.astype(jnp.float32) + d * wd + a * wa + wb, 0.0)
            return c2
        lax.fori_loop(0, CH, edge_body, 0)

    nphases = jnp.where(cid == SLOW_CID, SLOWW // PC, FASTW // PC)
    start = jnp.where(cid == SLOW_CID, sid * SLOWW, 16 * SLOWW + sid * FASTW)

    def phase_body(ph, c0):
        base = start + ph * PC
        pltpu.sync_copy(src_hbm.at[pl.ds(base, PC)], srcb)
        pltpu.sync_copy(dst_hbm.at[pl.ds(base, PC)], dstb)
        pltpu.sync_copy(attr_hbm.at[pl.ds(base, PC)], attrb)
        gather_issue(0, 0)

        def outer_body(tt, c):
            for b in range(2):
                t = tt * 2 + b
                ob = 1 - b

                @pl.when(t <= PC - 2)
                def _():
                    gather_issue(t + 1, ob)

                gather_wait(t, b)
                compute_chunk(t, b, msgb)
                pltpu.sync_copy(msgb, agg_sh.at[dstb.at[t]], add=True)
            return c
        lax.fori_loop(0, PC // 2, outer_body, 0)
        return c0
    lax.fori_loop(0, nphases, phase_body, 0)
    plsc.subcore_barrier()
    pltpu.sync_copy(agg_sh.at[pl.ds(sid * ROWS_PER_TILE, ROWS_PER_TILE)],
                    out_hbm.at[cid, pl.ds(sid * ROWS_PER_TILE, ROWS_PER_TILE)])


def _gine_aggregate(feat, pos16, src_pad, dst_pad, attr_pad, wpack):
    mesh = plsc.VectorSubcoreMesh(core_axis_name="c", subcore_axis_name="s")
    fn = functools.partial(
        pl.kernel,
        out_type=jax.ShapeDtypeStruct((2, NPAD, H), jnp.float32),
        mesh=mesh,
        scratch_types=[
            pltpu.VMEM((PC, CH), jnp.int32),
            pltpu.VMEM((PC, CH), jnp.int32),
            pltpu.VMEM((PC, CH), jnp.float32),
            pltpu.VMEM((CH + 16,), jnp.float32),
            pltpu.VMEM((CH + 16,), jnp.float32),
            pltpu.VMEM((CH, H // 2), jnp.int32),
            pltpu.VMEM((CH, H // 2), jnp.int32),
            pltpu.VMEM((CH, H), jnp.float32),
            pltpu.VMEM((CH, 16), jnp.float32),
            pltpu.VMEM((CH, 16), jnp.float32),
            pltpu.VMEM((CH, 16), jnp.float32),
            pltpu.VMEM((CH, 16), jnp.float32),
            pltpu.VMEM((3, H), jnp.float32),
            pltpu.VMEM_SHARED((NPAD, H), jnp.float32),
            pltpu.SemaphoreType.DMA,
            pltpu.SemaphoreType.DMA,
            pltpu.SemaphoreType.DMA,
            pltpu.SemaphoreType.DMA,
        ],
        compiler_params=pltpu.CompilerParams(needs_layout_passes=False, use_tc_tiling_on_sc=False),
    )(_gine_sc_body)
    fb16 = feat.astype(jnp.bfloat16).reshape(feat.shape[0], 4, 2, 16)
    lo = fb16[:, :, 0, :]
    hi = fb16[:, :, 1, :]
    packed = lax.bitcast_convert_type(
        jnp.stack([lo, hi], axis=-1), jnp.int32).reshape(feat.shape[0], H // 2)
    return fn(packed, pos16, src_pad.reshape(EPAD // CH, CH),
              dst_pad.reshape(EPAD // CH, CH),
              attr_pad.reshape(EPAD // CH, CH), wpack)


def _gine_mlp_kernel(feat_ref, agg_ref, w1_ref, b1_ref, w2_ref, b2_ref,
                     wp_ref, bp_ref, feat2_ref, pred_ref):
    f = feat_ref[...]
    z = f + agg_ref[0] + agg_ref[1]
    o = _dot(jnp.maximum(_dot(z, w1_ref[...]) + b1_ref[...], 0.0),
             w2_ref[...]) + b2_ref[...] + f
    feat2_ref[...] = o
    p = _dot(o, wp_ref[...]) + bp_ref[...]
    pred_ref[...] = 1.0 / (1.0 + jnp.exp(-p))


def _gine_mlp(feat, agg2, p):
    B = 400
    grid = (N // B,)
    w1, b1, w2, b2 = p['gine']
    wp, bp = p['pred']
    b1 = b1.reshape(1, H)
    b2 = b2.reshape(1, H)
    bp = bp.reshape(1, 1)
    full = lambda arr: pl.BlockSpec(arr.shape, lambda i: (0,) * arr.ndim)
    return pl.pallas_call(
        _gine_mlp_kernel,
        out_shape=(jax.ShapeDtypeStruct((N, H), jnp.float32),
                   jax.ShapeDtypeStruct((N, 1), jnp.float32)),
        grid=grid,
        in_specs=[pl.BlockSpec((B, H), lambda i: (i, 0)),
                  pl.BlockSpec((2, B, H), lambda i: (0, i, 0)),
                  full(w1), full(b1), full(w2), full(b2), full(wp), full(bp)],
        out_specs=(pl.BlockSpec((B, H), lambda i: (i, 0)),
                   pl.BlockSpec((B, 1), lambda i: (i, 0))),
    )(feat, agg2, w1, b1, w2, b2, wp, bp)


def _pro_kernel(pe_ref, w_ref, b_ref, g_ref, bl_ref, out_ref):
    ph = _dot(pe_ref[...], w_ref[...]) + b_ref[...]
    mu = jnp.mean(ph, axis=-1, keepdims=True)
    d = ph - mu
    var = jnp.mean(d * d, axis=-1, keepdims=True)
    out_ref[...] = d * lax.rsqrt(var + 1e-5) * g_ref[...] + bl_ref[...]


def _pro_out(pro_emb, p):
    B = 256
    w, b = p['prot_lin']
    g, bl = p['ln']
    b = b.reshape(1, H)
    g = g.reshape(1, H)
    bl = bl.reshape(1, H)
    full = lambda arr: pl.BlockSpec(arr.shape, lambda i: (0,) * arr.ndim)
    return pl.pallas_call(
        _pro_kernel,
        out_shape=jax.ShapeDtypeStruct((NPRO, H), jnp.float32),
        grid=(NPRO // B,),
        in_specs=[pl.BlockSpec((B, PDIM), lambda i: (i, 0)),
                  full(w), full(b), full(g), full(bl)],
        out_specs=pl.BlockSpec((B, H), lambda i: (i, 0)),
    )(pro_emb, w, b, g, bl)


def _cross_kernel(r_ref, w_ref, b_ref, out_ref):
    out_ref[...] = (1.0 / r_ref[...]) * w_ref[...] + b_ref[...]


def _cross_all(r_all, p):
    B = 2048
    w, b = p['cross_lin']
    b = b.reshape(1, H)
    R = r_all.shape[0]
    full = lambda arr: pl.BlockSpec(arr.shape, lambda i: (0,) * arr.ndim)
    return pl.pallas_call(
        _cross_kernel,
        out_shape=jax.ShapeDtypeStruct((R, H), jnp.float32),
        grid=(R // B,),
        in_specs=[pl.BlockSpec((B, 1), lambda i: (i, 0)), full(w), full(b)],
        out_specs=pl.BlockSpec((B, H), lambda i: (i, 0)),
    )(r_all, w, b)


def _merge_kernel(pe_ref, mp_ref, out_ref):
    out_ref[:, 0:EPRO] = pe_ref[...]
    out_ref[0:1, EPRO:EPRO + M] = mp_ref[0:1, :]
    out_ref[1:2, EPRO:EPRO + M] = mp_ref[1:2, :] + NPRO
    out_ref[0:1, EPRO + M:EPRO + 2 * M] = mp_ref[1:2, :] + NPRO
    out_ref[1:2, EPRO + M:EPRO + 2 * M] = mp_ref[0:1, :]


def _merge(pro_edge, mp):
    return pl.pallas_call(
        _merge_kernel,
        out_shape=jax.ShapeDtypeStruct((2, EPRO + 2 * M), jnp.int32),
    )(pro_edge, mp)


def kernel(params, pro_emb, pos, atom_cat, atom_dist, x, edge_index, edge_attr,
           merge_pro_vertex_edge, pro_edge, prot_dist, prot_ind):
    p = params
    feat = _node_encoder(atom_cat, atom_dist, x, p)

    pos16 = jnp.zeros((NPAD, 16), jnp.float32).at[:N, 0:3].set(pos)
    npadE = EPAD - E
    src_pad = jnp.concatenate([edge_index[0], jnp.zeros((npadE,), jnp.int32)])
    dst_pad = jnp.concatenate(
        [edge_index[1], N + (jnp.arange(npadE, dtype=jnp.int32) % (NPAD - N))])
    attr_pad = jnp.concatenate([edge_attr, jnp.zeros((npadE,), jnp.float32)])
    we, be = p['edge_lin']
    wpack = jnp.stack([we[0], we[1], be], 0)
    agg2 = _gine_aggregate(feat, pos16, src_pad, dst_pad, attr_pad, wpack)

    feat2, pred2d = _gine_mlp(feat, agg2, p)
    pred = pred2d[:, 0]

    pro_out = _pro_out(pro_emb, p)

    r_all = jnp.concatenate([prot_ind, prot_dist, prot_dist], axis=0)
    cross_all = _cross_all(r_all, p)

    merge = _merge(pro_edge, merge_pro_vertex_edge)
    return (pro_out, pred, feat2, merge, cross_all)


# speedup vs baseline: 1.1933x; 1.0176x over previous
import functools

import jax
import jax.numpy as jnp
from jax import lax
from jax.experimental import pallas as pl
from jax.experimental.pallas import tpu as pltpu
from jax.experimental.pallas import tpu_sc as plsc

N = 10000
K = 16
E = 160000
NPRO = 2048
EPRO = 32768
M = 65536
H = 128
VOCAB = 64
PDIM = 1280

NPAD = 10240
CH = 128
NWORK = 32
EPAD = 163840
CPW = EPAD // (NWORK * CH)
ROWS_PER_TILE = NPAD // 16
OUT_ROWS_PER_TILE = N // 16


def _dot(a, b):
    return jnp.dot(a, b, preferred_element_type=jnp.float32)


def _node_kernel(ac_ref, ad_ref, x_ref,
                 emb_ref, w1a_ref, b1a_ref, w2a_ref, b2a_ref,
                 w1b_ref, b1b_ref, w2b_ref, b2b_ref,
                 w1c_ref, b1c_ref, w2c_ref, b2c_ref,
                 wf_ref, bf_ref, feat_ref, pack_ref):
    B = ac_ref.shape[0]
    T = _dot(emb_ref[...], w1a_ref[0:32, :])
    w1last = w1a_ref[32:33, :]
    acc = jnp.zeros((B, H), jnp.float32)
    iot = lax.broadcasted_iota(jnp.int32, (B, VOCAB), 1)
    for k in range(K):
        ack = ac_ref[:, k:k + 1]
        oh = (ack == iot).astype(jnp.float32)
        l1 = _dot(oh, T) + (1.0 / ad_ref[:, k:k + 1]) * w1last + b1a_ref[...]
        acc = acc + jnp.maximum(l1, 0.0)
    h = _dot(acc, w2a_ref[...]) + K * b2a_ref[...]
    h = _dot(jnp.maximum(_dot(h, w1b_ref[...]) + b1b_ref[...], 0.0),
             w2b_ref[...]) + b2b_ref[...]
    geom = x_ref[:, 0:1]
    l = jnp.maximum(_dot(h, w1c_ref[0:H, :]) + geom * w1c_ref[H:H + 1, :]
                    + b1c_ref[...], 0.0)
    l = _dot(l, w2c_ref[...]) + b2c_ref[...]
    feat = _dot(l, wf_ref[...]) + bf_ref[...]
    feat_ref[...] = feat
    xi = lax.bitcast_convert_type(feat, jnp.int32)
    r = xi + 0x7FFF + (lax.shift_right_logical(xi, 16) & 1)
    bits = lax.shift_right_logical(r, 16)
    pack_ref[...] = bits[:, 0:64] | lax.shift_left(bits[:, 64:H], 16)


def _node_encoder(atom_cat, atom_dist, x, p):
    B = 400
    grid = (N // B,)
    full = lambda arr: pl.BlockSpec(arr.shape, lambda i: (0,) * arr.ndim)
    row = lambda c: pl.BlockSpec((B, c), lambda i: (i, 0))
    emb = p['emb']
    w1a, b1a, w2a, b2a = p['atom_a']
    w1b, b1b, w2b, b2b = p['atom_b']
    w1c, b1c, w2c, b2c = p['chem']
    wf, bf = p['feat_scale']
    b1a, b2a, b1b, b2b, b1c, b2c, bf = (
        v.reshape(1, H) for v in (b1a, b2a, b1b, b2b, b1c, b2c, bf))
    args = (atom_cat, atom_dist, x, emb, w1a, b1a, w2a, b2a,
            w1b, b1b, w2b, b2b, w1c, b1c, w2c, b2c, wf, bf)
    in_specs = [row(K), row(K), row(x.shape[1])] + [full(a) for a in args[3:]]
    return pl.pallas_call(
        _node_kernel,
        out_shape=(jax.ShapeDtypeStruct((N, H), jnp.float32),
                   jax.ShapeDtypeStruct((N, H // 2), jnp.int32)),
        grid=grid,
        in_specs=in_specs,
        out_specs=(pl.BlockSpec((B, H), lambda i: (i, 0)),
                   pl.BlockSpec((B, H // 2), lambda i: (i, 0))),
    )(*args)


NPHASE = 5
PC = CPW // NPHASE
SLOW_CID = 1
SLOWW = 40
FASTW = 40


def _gine_sc_body(feat_hbm, pos16_hbm, src_hbm, dst_hbm, attr_hbm, wpack_hbm,
                  out_hbm,
                  srcb, dstb, attrb, distb, attr1d,
                  featb0, featb1, msgb, spos0, spos1, dpos0, dpos1,
                  wpk_v, agg_sh,
                  gsem0, gsem1, ssem0, ssem1):
    cid = lax.axis_index("c")
    sid = lax.axis_index("s")
    wid = sid * 2 + cid
    featbs = (featb0, featb1)
    sposs = (spos0, spos1)
    dposs = (dpos0, dpos1)
    gsems = (gsem0, gsem1)
    pltpu.sync_copy(wpack_hbm, wpk_v)

    def zrow(i, c):
        for j in range(8):
            msgb[i, pl.ds(j * 16, 16)] = jnp.zeros((16,), jnp.float32)
        return c
    lax.fori_loop(0, CH, zrow, 0)
    for z in range(ROWS_PER_TILE // CH):
        pltpu.sync_copy(msgb, agg_sh.at[pl.ds(sid * ROWS_PER_TILE + z * CH, CH)])
    plsc.subcore_barrier()

    wvecs = [(wpk_v[0, pl.ds(j * 16, 16)],
              wpk_v[1, pl.ds(j * 16, 16)],
              wpk_v[2, pl.ds(j * 16, 16)]) for j in range(8)]
    lane = lax.iota(jnp.int32, 16)
    zc = lane * 0
    oc = zc + 1
    tc = zc + 2

    def gather_issue(t, b):
        pltpu.async_copy(feat_hbm.at[srcb.at[t]], featbs[b], gsems[b])
        pltpu.async_copy(pos16_hbm.at[srcb.at[t]], sposs[b], gsems[b])
        pltpu.async_copy(pos16_hbm.at[dstb.at[t]], dposs[b], gsems[b])

    def gather_wait(t, b):
        pltpu.make_async_copy(feat_hbm.at[srcb.at[t]], featbs[b],
                              gsems[b]).wait()
        pltpu.make_async_copy(pos16_hbm.at[srcb.at[t]], sposs[b],
                              gsems[b]).wait()
        pltpu.make_async_copy(pos16_hbm.at[dstb.at[t]], dposs[b],
                              gsems[b]).wait()

    def compute_chunk(t, b, msgb):
        fb = featbs[b]
        sp = sposs[b]
        dp = dposs[b]
        for i in range(8):
            sl16 = pl.ds(i * 16, 16)
            rows = lane + i * 16
            dx = (plsc.load_gather(sp, [rows, zc])
                  - plsc.load_gather(dp, [rows, zc]))
            dy = (plsc.load_gather(sp, [rows, oc])
                  - plsc.load_gather(dp, [rows, oc]))
            dz = (plsc.load_gather(sp, [rows, tc])
                  - plsc.load_gather(dp, [rows, tc]))
            r2 = dx * dx + dy * dy + dz * dz
            bi = plsc.bitcast(r2, jnp.int32)
            y = plsc.bitcast(jnp.int32(0x1FBD1DF5)
                             + lax.shift_right_logical(bi, 1), jnp.float32)
            y = 0.5 * (y + r2 / y)
            y = 0.5 * (y + r2 / y)
            y = 0.5 * (y + r2 / y)
            distb[sl16] = jnp.where(r2 > 0.0, y, 0.0)
            attr1d[sl16] = attrb[t, sl16]
        def edge_body(e, c2):
            d = distb[pl.ds(e, 16)][0]
            a = attr1d[pl.ds(e, 16)][0]
            for q in range(4):
                w = fb[e, pl.ds(q * 16, 16)]
                lo, hi = plsc.unpack(plsc.bitcast(w, jnp.bfloat16),
                                     format=plsc.PackFormat.INTERLEAVED)
                for h, v in ((q, lo), (q + 4, hi)):
                    sl = pl.ds(h * 16, 16)
                    wd, wa, wb = wvecs[h]
                    msgb[e, sl] = jnp.maximum(
                        v.astype(jnp.float32) + d * wd + a * wa + wb, 0.0)
            return c2
        lax.fori_loop(0, CH, edge_body, 0)

    nphases = jnp.where(cid == SLOW_CID, SLOWW // PC, FASTW // PC)
    start = jnp.where(cid == SLOW_CID, sid * SLOWW, 16 * SLOWW + sid * FASTW)

    def phase_body(ph, c0):
        base = start + ph * PC
        pltpu.sync_copy(src_hbm.at[pl.ds(base, PC)], srcb)
        pltpu.sync_copy(dst_hbm.at[pl.ds(base, PC)], dstb)
        pltpu.sync_copy(attr_hbm.at[pl.ds(base, PC)], attrb)
        gather_issue(0, 0)

        def outer_body(tt, c):
            for b in range(2):
                t = tt * 2 + b
                ob = 1 - b

                @pl.when(t <= PC - 2)
                def _():
                    gather_issue(t + 1, ob)

                gather_wait(t, b)
                compute_chunk(t, b, msgb)
                pltpu.sync_copy(msgb, agg_sh.at[dstb.at[t]], add=True)
            return c
        lax.fori_loop(0, PC // 2, outer_body, 0)
        return c0
    lax.fori_loop(0, nphases, phase_body, 0)
    plsc.subcore_barrier()
    pltpu.sync_copy(agg_sh.at[pl.ds(sid * ROWS_PER_TILE, ROWS_PER_TILE)],
                    out_hbm.at[cid, pl.ds(sid * ROWS_PER_TILE, ROWS_PER_TILE)])


def _gine_aggregate(packed, pos16, src_pad, dst_pad, attr_pad, wpack):
    mesh = plsc.VectorSubcoreMesh(core_axis_name="c", subcore_axis_name="s")
    fn = functools.partial(
        pl.kernel,
        out_type=jax.ShapeDtypeStruct((2, NPAD, H), jnp.float32),
        mesh=mesh,
        scratch_types=[
            pltpu.VMEM((PC, CH), jnp.int32),
            pltpu.VMEM((PC, CH), jnp.int32),
            pltpu.VMEM((PC, CH), jnp.float32),
            pltpu.VMEM((CH + 16,), jnp.float32),
            pltpu.VMEM((CH + 16,), jnp.float32),
            pltpu.VMEM((CH, H // 2), jnp.int32),
            pltpu.VMEM((CH, H // 2), jnp.int32),
            pltpu.VMEM((CH, H), jnp.float32),
            pltpu.VMEM((CH, 16), jnp.float32),
            pltpu.VMEM((CH, 16), jnp.float32),
            pltpu.VMEM((CH, 16), jnp.float32),
            pltpu.VMEM((CH, 16), jnp.float32),
            pltpu.VMEM((3, H), jnp.float32),
            pltpu.VMEM_SHARED((NPAD, H), jnp.float32),
            pltpu.SemaphoreType.DMA,
            pltpu.SemaphoreType.DMA,
            pltpu.SemaphoreType.DMA,
            pltpu.SemaphoreType.DMA,
        ],
        compiler_params=pltpu.CompilerParams(needs_layout_passes=False, use_tc_tiling_on_sc=False),
    )(_gine_sc_body)
    return fn(packed, pos16, src_pad.reshape(EPAD // CH, CH),
              dst_pad.reshape(EPAD // CH, CH),
              attr_pad.reshape(EPAD // CH, CH), wpack)


def _gine_mlp_kernel(feat_ref, agg_ref, w1_ref, b1_ref, w2_ref, b2_ref,
                     wp_ref, bp_ref, feat2_ref, pred_ref):
    f = feat_ref[...]
    z = f + agg_ref[0] + agg_ref[1]
    o = _dot(jnp.maximum(_dot(z, w1_ref[...]) + b1_ref[...], 0.0),
             w2_ref[...]) + b2_ref[...] + f
    feat2_ref[...] = o
    p = _dot(o, wp_ref[...]) + bp_ref[...]
    pred_ref[...] = 1.0 / (1.0 + jnp.exp(-p))


def _gine_mlp(feat, agg2, p):
    B = 400
    grid = (N // B,)
    w1, b1, w2, b2 = p['gine']
    wp, bp = p['pred']
    b1 = b1.reshape(1, H)
    b2 = b2.reshape(1, H)
    bp = bp.reshape(1, 1)
    full = lambda arr: pl.BlockSpec(arr.shape, lambda i: (0,) * arr.ndim)
    return pl.pallas_call(
        _gine_mlp_kernel,
        out_shape=(jax.ShapeDtypeStruct((N, H), jnp.float32),
                   jax.ShapeDtypeStruct((N, 1), jnp.float32)),
        grid=grid,
        in_specs=[pl.BlockSpec((B, H), lambda i: (i, 0)),
                  pl.BlockSpec((2, B, H), lambda i: (0, i, 0)),
                  full(w1), full(b1), full(w2), full(b2), full(wp), full(bp)],
        out_specs=(pl.BlockSpec((B, H), lambda i: (i, 0)),
                   pl.BlockSpec((B, 1), lambda i: (i, 0))),
    )(feat, agg2, w1, b1, w2, b2, wp, bp)


def _pro_kernel(pe_ref, w_ref, b_ref, g_ref, bl_ref, out_ref):
    ph = _dot(pe_ref[...], w_ref[...]) + b_ref[...]
    mu = jnp.mean(ph, axis=-1, keepdims=True)
    d = ph - mu
    var = jnp.mean(d * d, axis=-1, keepdims=True)
    out_ref[...] = d * lax.rsqrt(var + 1e-5) * g_ref[...] + bl_ref[...]


def _pro_out(pro_emb, p):
    B = 256
    w, b = p['prot_lin']
    g, bl = p['ln']
    b = b.reshape(1, H)
    g = g.reshape(1, H)
    bl = bl.reshape(1, H)
    full = lambda arr: pl.BlockSpec(arr.shape, lambda i: (0,) * arr.ndim)
    return pl.pallas_call(
        _pro_kernel,
        out_shape=jax.ShapeDtypeStruct((NPRO, H), jnp.float32),
        grid=(NPRO // B,),
        in_specs=[pl.BlockSpec((B, PDIM), lambda i: (i, 0)),
                  full(w), full(b), full(g), full(bl)],
        out_specs=pl.BlockSpec((B, H), lambda i: (i, 0)),
    )(pro_emb, w, b, g, bl)


def _cross_kernel(r_ref, w_ref, b_ref, out_ref):
    out_ref[...] = (1.0 / r_ref[...]) * w_ref[...] + b_ref[...]


def _cross_all(r_all, p):
    B = 2048
    w, b = p['cross_lin']
    b = b.reshape(1, H)
    R = r_all.shape[0]
    full = lambda arr: pl.BlockSpec(arr.shape, lambda i: (0,) * arr.ndim)
    return pl.pallas_call(
        _cross_kernel,
        out_shape=jax.ShapeDtypeStruct((R, H), jnp.float32),
        grid=(R // B,),
        in_specs=[pl.BlockSpec((B, 1), lambda i: (i, 0)), full(w), full(b)],
        out_specs=pl.BlockSpec((B, H), lambda i: (i, 0)),
    )(r_all, w, b)


def _merge_kernel(pe_ref, mp_ref, out_ref):
    out_ref[:, 0:EPRO] = pe_ref[...]
    out_ref[0:1, EPRO:EPRO + M] = mp_ref[0:1, :]
    out_ref[1:2, EPRO:EPRO + M] = mp_ref[1:2, :] + NPRO
    out_ref[0:1, EPRO + M:EPRO + 2 * M] = mp_ref[1:2, :] + NPRO
    out_ref[1:2, EPRO + M:EPRO + 2 * M] = mp_ref[0:1, :]


def _merge(pro_edge, mp):
    return pl.pallas_call(
        _merge_kernel,
        out_shape=jax.ShapeDtypeStruct((2, EPRO + 2 * M), jnp.int32),
    )(pro_edge, mp)


def kernel(params, pro_emb, pos, atom_cat, atom_dist, x, edge_index, edge_attr,
           merge_pro_vertex_edge, pro_edge, prot_dist, prot_ind):
    p = params
    feat, packed = _node_encoder(atom_cat, atom_dist, x, p)

    pos16 = jnp.zeros((NPAD, 16), jnp.float32).at[:N, 0:3].set(pos)
    npadE = EPAD - E
    src_pad = jnp.concatenate([edge_index[0], jnp.zeros((npadE,), jnp.int32)])
    dst_pad = jnp.concatenate(
        [edge_index[1], N + (jnp.arange(npadE, dtype=jnp.int32) % (NPAD - N))])
    attr_pad = jnp.concatenate([edge_attr, jnp.zeros((npadE,), jnp.float32)])
    we, be = p['edge_lin']
    wpack = jnp.stack([we[0], we[1], be], 0)
    agg2 = _gine_aggregate(packed, pos16, src_pad, dst_pad, attr_pad, wpack)

    feat2, pred2d = _gine_mlp(feat, agg2, p)
    pred = pred2d[:, 0]

    pro_out = _pro_out(pro_emb, p)

    r_all = jnp.concatenate([prot_ind, prot_dist, prot_dist], axis=0)
    cross_all = _cross_all(r_all, p)

    merge = _merge(pro_edge, merge_pro_vertex_edge)
    return (pro_out, pred, feat2, merge, cross_all)
